# baseline (speedup 1.0000x reference)
import functools

import jax
import jax.numpy as jnp
from jax.experimental import pallas as pl
from jax.experimental.pallas import tpu as pltpu

B = 2
V = 32
NI = 16
NW = 16
H = 32
K = 8
N_NODES = 16
E = 12
BN_EPS = 1e-5
BN_SCALE = (1.0 + BN_EPS) ** -0.5

GPS = 8
SG_PER_STEP = 128
GB = GPS * SG_PER_STEP
D_STEP = GB * B
LN = GPS * N_NODES
LE = GPS * E
DSG = GPS * B
LANES = SG_PER_STEP * 128


def _pad8(n):
    return ((n + 7) // 8) * 8


def _pack_slab(entries):
    off = {}
    row = 0
    pieces = []
    for name, a in entries:
        h, w = a.shape
        hp = _pad8(h)
        off[name] = (row, h, w)
        row += hp
        pieces.append(jnp.pad(a.astype(jnp.float32),
                              ((0, hp - h), (0, 128 - w))))
    return jnp.concatenate(pieces, axis=0), off


def _fused_kernel(off, slab_ref, idxx_ref, src_ref, idxw_ref, xb_ref,
                  dst_ref, ew_ref, docT_ref, theta_ref, scal_ref):
    f32 = jnp.float32
    bf16 = jnp.bfloat16
    i32 = jnp.int32
    iota = jax.lax.broadcasted_iota
    nt = (((1,), (1,)), ((), ()))

    def W(name):
        r, h, w = off[name]
        return slab_ref[r:r + h, 0:w]

    def bdot(a, b):
        return jnp.dot(a, b, preferred_element_type=f32)

    idxx = idxx_ref[0]
    oh_x = (iota(i32, (V, LANES), 0) == idxx).astype(f32)
    xT = jnp.dot(W('wvT'), oh_x, preferred_element_type=f32)

    eoff = iota(i32, (1, 128), 1) // E * N_NODES
    ew = ew_ref[0]
    iota_n = iota(i32, (LN, 128), 0)
    agg_parts = []
    for sg in range(SG_PER_STEP):
        lo, hi = sg * 128, (sg + 1) * 128
        gsrc = src_ref[0][:, lo:hi] + eoff
        msrc = (iota_n == gsrc).astype(f32)
        x_srcT = jnp.dot(xT[:, lo:hi], msrc,
                         preferred_element_type=f32) * ew[:, lo:hi]
        gdst = dst_ref[0][:, lo:hi] + eoff
        mdstT = (iota_n == gdst).astype(f32)
        agg_parts.append(
            jax.lax.dot_general(x_srcT, mdstT, nt,
                                preferred_element_type=f32))
    aggT = jnp.concatenate(agg_parts, axis=1)
    aggT = aggT + idxw_ref[0] * xT

    ax = jnp.concatenate([aggT, xT], axis=0)
    gnnT = bdot(W('Wrr'), ax) + W('b_gnn')
    enc1T = jnp.tanh(gnnT * BN_SCALE)
    ex = jnp.concatenate([enc1T, xT], axis=0)
    pre = bdot(W('W12'), ex) + W('b12')
    gT = jax.nn.sigmoid(pre[0:H]) * jnp.tanh(pre[H:2 * H])

    doff = iota(i32, (1, 128), 1) // N_NODES * B
    enc2_parts = []
    for sg in range(SG_PER_STEP):
        lo, hi = sg * 128, (sg + 1) * 128
        gdoc = xb_ref[0][:, lo:hi] + doff
        mselT = (iota(i32, (DSG, 128), 0) == gdoc).astype(f32)
        enc2_parts.append(
            jax.lax.dot_general(gT[:, lo:hi], mselT, nt,
                                preferred_element_type=f32))
    enc2T = jnp.concatenate(enc2_parts, axis=1)

    gm = bdot(W('Wg'), enc2T) + W('bg')
    post_mean = gm[0:K] * BN_SCALE
    post_logvar = gm[K:2 * K]

    docT = docT_ref[...]
    h1 = jnp.tanh(bdot(W('We1'), docT) + W('be1'))
    h2 = jnp.tanh(bdot(W('We2'), h1) + W('be2'))
    pp = bdot(W('Wp'), h2) + W('bp')
    prior_mean = pp[0:K]
    prior_logvar = pp[K:2 * K]

    td = bdot(W('Wdec'), prior_mean) + W('bdec')
    e = jnp.exp(td - jnp.max(td, axis=0, keepdims=True))
    theta = e / jnp.sum(e, axis=0, keepdims=True)
    recon = jnp.dot(W('betaT'), theta, preferred_element_type=f32)
    nl = -jnp.sum(docT * jnp.log(recon + 1e-10), axis=0, keepdims=True)

    post_var = jnp.exp(post_logvar)
    prior_var = jnp.exp(prior_logvar)
    kl1 = 0.5 * jnp.sum(
        prior_logvar - post_logvar
        + (post_var + (post_mean - prior_mean) ** 2) / prior_var - 1.0,
        axis=0, keepdims=True)
    kl2 = -0.5 * jnp.sum(1.0 - post_mean ** 2 + post_logvar - post_var,
                         axis=0, keepdims=True)

    quad = jnp.concatenate([nl + kl1 + kl2, nl, kl1, kl2], axis=0)
    pool = (iota(i32, (D_STEP, GB), 0) // B
            == iota(i32, (D_STEP, GB), 1)).astype(f32)
    scal_ref[...] = jnp.dot(quad, pool,
                            preferred_element_type=f32) * (1.0 / B)
    theta_ref[...] = theta


def kernel(word_vec, gnn_wrel, gnn_brel, gnn_wroot, enc2_fc1_w, enc2_fc1_b,
           enc2_fc2_w, enc2_fc2_b, ge_mean_w, ge_mean_b, ge_logvar_w,
           ge_logvar_b, enc1_fc_w, enc1_fc_b, enc2_fc_w, enc2_fc_b,
           mean_fc_w, mean_fc_b, logvar_fc_w, logvar_fc_b, decoder_w,
           decoder_b, topic_vec, idx_x, idx_w, x_batch, edge_index,
           edge_w, doc_input):
    f32 = jnp.float32
    i32 = jnp.int32

    batched = doc_input.ndim == 3
    if not batched:
        idx_x, idx_w, x_batch = idx_x[None], idx_w[None], x_batch[None]
        edge_index, edge_w, doc_input = (edge_index[None], edge_w[None],
                                         doc_input[None])
    G = doc_input.shape[0]
    Gp = ((G + GB - 1) // GB) * GB
    steps = Gp // GB

    beta = jax.nn.softmax(
        jnp.asarray(topic_vec, f32) @ jnp.asarray(word_vec, f32).T, axis=-1)
    w1 = jnp.asarray(enc2_fc1_w, f32)
    w2 = jnp.asarray(enc2_fc2_w, f32)
    entries = [
        ('wvT', jnp.asarray(word_vec, f32).T),
        ('Wrr', jnp.concatenate([jnp.asarray(gnn_wrel, f32),
                                 jnp.asarray(gnn_wroot, f32)], axis=1)),
        ('b_gnn', jnp.asarray(gnn_brel, f32).T),
        ('W12', jnp.concatenate(
            [jnp.concatenate([w1[:, :NW], w2[:, :NW]], axis=0),
             jnp.concatenate([w1[:, NW:], w2[:, NW:]], axis=0)], axis=1)),
        ('b12', jnp.concatenate([jnp.asarray(enc2_fc1_b, f32),
                                 jnp.asarray(enc2_fc2_b, f32)], axis=1).T),
        ('Wg', jnp.concatenate([jnp.asarray(ge_mean_w, f32),
                                jnp.asarray(ge_logvar_w, f32)], axis=0)),
        ('bg', jnp.concatenate([jnp.asarray(ge_mean_b, f32),
                                jnp.asarray(ge_logvar_b, f32)], axis=1).T),
        ('We1', jnp.asarray(enc1_fc_w, f32)),
        ('be1', jnp.asarray(enc1_fc_b, f32).T),
        ('We2', jnp.asarray(enc2_fc_w, f32)),
        ('be2', jnp.asarray(enc2_fc_b, f32).T),
        ('Wp', jnp.concatenate([jnp.asarray(mean_fc_w, f32),
                                jnp.asarray(logvar_fc_w, f32)], axis=0)),
        ('bp', jnp.concatenate([jnp.asarray(mean_fc_b, f32),
                                jnp.asarray(logvar_fc_b, f32)], axis=1).T),
        ('Wdec', jnp.asarray(decoder_w, f32)),
        ('bdec', jnp.asarray(decoder_b, f32).T),
        ('betaT', beta.T),
    ]
    slab, off = _pack_slab(entries)
    w_rows = slab.shape[0]

    def padg(a, fill=0):
        if Gp == G:
            return a
        pad_shape = (Gp - G,) + a.shape[1:]
        return jnp.concatenate([a, jnp.full(pad_shape, fill, a.dtype)], axis=0)

    def rows_n(a):
        return a.reshape(steps, 1, LANES)

    def rows_e(a):
        a = a.reshape(Gp // GPS, LE)
        a = jnp.concatenate(
            [a, jnp.zeros((Gp // GPS, 128 - LE), a.dtype)], axis=1)
        return a.reshape(steps, 1, LANES)

    idx_x_i = jnp.asarray(idx_x, i32)
    idxx3 = rows_n(padg(idx_x_i))
    src3 = rows_e(padg(jnp.asarray(edge_index[:, 0, :], i32)))
    idxw3 = rows_n(padg(jnp.asarray(idx_w, f32)))
    xb3 = rows_n(padg(jnp.asarray(x_batch, i32)))
    dst3 = rows_e(padg(jnp.asarray(edge_index[:, 1, :], i32)))
    ew3 = rows_e(padg(jnp.asarray(edge_w, f32)))
    docT = padg(jnp.asarray(doc_input, f32)).reshape(Gp * B, V).T

    kern = functools.partial(_fused_kernel, off)
    thetaT, scal = pl.pallas_call(
        kern,
        out_shape=[jax.ShapeDtypeStruct((K, Gp * B), f32),
                   jax.ShapeDtypeStruct((4, Gp), f32)],
        grid_spec=pltpu.PrefetchScalarGridSpec(
            num_scalar_prefetch=0,
            grid=(steps,),
            in_specs=[
                pl.BlockSpec((w_rows, 128), lambda g: (0, 0)),
                pl.BlockSpec((1, 1, LANES), lambda g: (g, 0, 0)),
                pl.BlockSpec((1, 1, LANES), lambda g: (g, 0, 0)),
                pl.BlockSpec((1, 1, LANES), lambda g: (g, 0, 0)),
                pl.BlockSpec((1, 1, LANES), lambda g: (g, 0, 0)),
                pl.BlockSpec((1, 1, LANES), lambda g: (g, 0, 0)),
                pl.BlockSpec((1, 1, LANES), lambda g: (g, 0, 0)),
                pl.BlockSpec((V, D_STEP), lambda g: (0, g)),
            ],
            out_specs=[
                pl.BlockSpec((K, D_STEP), lambda g: (0, g)),
                pl.BlockSpec((4, GB), lambda g: (0, g)),
            ]),
        compiler_params=pltpu.CompilerParams(
            dimension_semantics=("parallel",)),
    )(slab, idxx3, src3, idxw3, xb3, dst3, ew3, docT)

    theta = thetaT.T.reshape(Gp, B, K)[:G]
    loss = scal[0, :G]
    outputs = {'loss': loss,
               'recon_word': scal[1, :G],
               'KL1': scal[2, :G],
               'KL2': scal[3, :G],
               'recon_structure': jnp.zeros_like(loss)}
    if not batched:
        outputs = jax.tree_util.tree_map(lambda a: a[0], outputs)
        theta, loss = theta[0], loss[0]
    return outputs, theta, loss


# speedup vs baseline: 1.4097x; 1.4097x over previous
import functools

import jax
import jax.numpy as jnp
from jax.experimental import pallas as pl
from jax.experimental.pallas import tpu as pltpu

B = 2
V = 32
NI = 16
NW = 16
H = 32
K = 8
N_NODES = 16
E = 12
BN_EPS = 1e-5
BN_SCALE = (1.0 + BN_EPS) ** -0.5

GPS = 8
SG_PER_STEP = 256
GB = GPS * SG_PER_STEP
D_STEP = GB * B
LN = GPS * N_NODES
LE = GPS * E
DSG = GPS * B
LANES = SG_PER_STEP * 128


def _pad8(n):
    return ((n + 7) // 8) * 8


def _pack_slab(entries):
    off = {}
    row = 0
    pieces = []
    for name, a in entries:
        h, w = a.shape
        hp = _pad8(h)
        off[name] = (row, h, w)
        row += hp
        pieces.append(jnp.pad(a.astype(jnp.float32),
                              ((0, hp - h), (0, 128 - w))))
    return jnp.concatenate(pieces, axis=0), off


def _fused_kernel(off, slab_ref, idxx_ref, idxs_ref, idxw_ref, xb_ref,
                  dst_ref, ew_ref, docT_ref, theta_ref, scal_ref):
    f32 = jnp.float32
    bf16 = jnp.bfloat16
    i32 = jnp.int32
    iota = jax.lax.broadcasted_iota
    nt = (((1,), (1,)), ((), ()))

    def W(name):
        r, h, w = off[name]
        return slab_ref[r:r + h, 0:w]

    def bdot(a, b):
        return jnp.dot(a, b, preferred_element_type=f32)

    idxx = idxx_ref[0]
    oh_x = (iota(i32, (V, LANES), 0) == idxx).astype(f32)
    xT = jnp.dot(W('wvT'), oh_x, preferred_element_type=f32)

    idxs = idxs_ref[0]
    oh_s = (iota(i32, (V, LANES), 0) == idxs).astype(f32)
    x_srcT = jnp.dot(W('wvT'), oh_s, preferred_element_type=f32)
    x_srcT = x_srcT * ew_ref[0]

    eoff = iota(i32, (1, 128), 1) // E * N_NODES
    agg_parts = []
    for sg in range(SG_PER_STEP):
        lo, hi = sg * 128, (sg + 1) * 128
        gdst = dst_ref[0][:, lo:hi] + eoff
        mdstT = (iota(i32, (LN, 128), 0) == gdst).astype(f32)
        agg_parts.append(
            jax.lax.dot_general(x_srcT[:, lo:hi], mdstT, nt,
                                preferred_element_type=f32))
    aggT = jnp.concatenate(agg_parts, axis=1)
    aggT = aggT + idxw_ref[0] * xT

    ax = jnp.concatenate([aggT, xT], axis=0)
    gnnT = bdot(W('Wrr'), ax) + W('b_gnn')
    enc1T = jnp.tanh(gnnT * BN_SCALE)
    ex = jnp.concatenate([enc1T, xT], axis=0)
    pre = bdot(W('W12'), ex) + W('b12')
    gT = jax.nn.sigmoid(pre[0:H]) * jnp.tanh(pre[H:2 * H])

    doff = iota(i32, (1, 128), 1) // N_NODES * B
    enc2_parts = []
    for sg in range(SG_PER_STEP):
        lo, hi = sg * 128, (sg + 1) * 128
        gdoc = xb_ref[0][:, lo:hi] + doff
        mselT = (iota(i32, (DSG, 128), 0) == gdoc).astype(f32)
        enc2_parts.append(
            jax.lax.dot_general(gT[:, lo:hi], mselT, nt,
                                preferred_element_type=f32))
    enc2T = jnp.concatenate(enc2_parts, axis=1)

    gm = bdot(W('Wg'), enc2T) + W('bg')
    post_mean = gm[0:K] * BN_SCALE
    post_logvar = gm[K:2 * K]

    docT = docT_ref[...]
    h1 = jnp.tanh(bdot(W('We1'), docT) + W('be1'))
    h2 = jnp.tanh(bdot(W('We2'), h1) + W('be2'))
    pp = bdot(W('Wp'), h2) + W('bp')
    prior_mean = pp[0:K]
    prior_logvar = pp[K:2 * K]

    td = bdot(W('Wdec'), prior_mean) + W('bdec')
    e = jnp.exp(td - jnp.max(td, axis=0, keepdims=True))
    theta = e / jnp.sum(e, axis=0, keepdims=True)
    recon = jnp.dot(W('betaT'), theta, preferred_element_type=f32)
    nl = -jnp.sum(docT * jnp.log(recon + 1e-10), axis=0, keepdims=True)

    post_var = jnp.exp(post_logvar)
    prior_var = jnp.exp(prior_logvar)
    kl1 = 0.5 * jnp.sum(
        prior_logvar - post_logvar
        + (post_var + (post_mean - prior_mean) ** 2) / prior_var - 1.0,
        axis=0, keepdims=True)
    kl2 = -0.5 * jnp.sum(1.0 - post_mean ** 2 + post_logvar - post_var,
                         axis=0, keepdims=True)

    quad = jnp.concatenate([nl + kl1 + kl2, nl, kl1, kl2], axis=0)
    pool = (iota(i32, (D_STEP, GB), 0) // B
            == iota(i32, (D_STEP, GB), 1)).astype(f32)
    scal_ref[...] = jnp.dot(quad, pool,
                            preferred_element_type=f32) * (1.0 / B)
    theta_ref[...] = theta


def kernel(word_vec, gnn_wrel, gnn_brel, gnn_wroot, enc2_fc1_w, enc2_fc1_b,
           enc2_fc2_w, enc2_fc2_b, ge_mean_w, ge_mean_b, ge_logvar_w,
           ge_logvar_b, enc1_fc_w, enc1_fc_b, enc2_fc_w, enc2_fc_b,
           mean_fc_w, mean_fc_b, logvar_fc_w, logvar_fc_b, decoder_w,
           decoder_b, topic_vec, idx_x, idx_w, x_batch, edge_index,
           edge_w, doc_input):
    f32 = jnp.float32
    i32 = jnp.int32

    batched = doc_input.ndim == 3
    if not batched:
        idx_x, idx_w, x_batch = idx_x[None], idx_w[None], x_batch[None]
        edge_index, edge_w, doc_input = (edge_index[None], edge_w[None],
                                         doc_input[None])
    G = doc_input.shape[0]
    Gp = ((G + GB - 1) // GB) * GB
    steps = Gp // GB

    beta = jax.nn.softmax(
        jnp.asarray(topic_vec, f32) @ jnp.asarray(word_vec, f32).T, axis=-1)
    w1 = jnp.asarray(enc2_fc1_w, f32)
    w2 = jnp.asarray(enc2_fc2_w, f32)
    entries = [
        ('wvT', jnp.asarray(word_vec, f32).T),
        ('Wrr', jnp.concatenate([jnp.asarray(gnn_wrel, f32),
                                 jnp.asarray(gnn_wroot, f32)], axis=1)),
        ('b_gnn', jnp.asarray(gnn_brel, f32).T),
        ('W12', jnp.concatenate(
            [jnp.concatenate([w1[:, :NW], w2[:, :NW]], axis=0),
             jnp.concatenate([w1[:, NW:], w2[:, NW:]], axis=0)], axis=1)),
        ('b12', jnp.concatenate([jnp.asarray(enc2_fc1_b, f32),
                                 jnp.asarray(enc2_fc2_b, f32)], axis=1).T),
        ('Wg', jnp.concatenate([jnp.asarray(ge_mean_w, f32),
                                jnp.asarray(ge_logvar_w, f32)], axis=0)),
        ('bg', jnp.concatenate([jnp.asarray(ge_mean_b, f32),
                                jnp.asarray(ge_logvar_b, f32)], axis=1).T),
        ('We1', jnp.asarray(enc1_fc_w, f32)),
        ('be1', jnp.asarray(enc1_fc_b, f32).T),
        ('We2', jnp.asarray(enc2_fc_w, f32)),
        ('be2', jnp.asarray(enc2_fc_b, f32).T),
        ('Wp', jnp.concatenate([jnp.asarray(mean_fc_w, f32),
                                jnp.asarray(logvar_fc_w, f32)], axis=0)),
        ('bp', jnp.concatenate([jnp.asarray(mean_fc_b, f32),
                                jnp.asarray(logvar_fc_b, f32)], axis=1).T),
        ('Wdec', jnp.asarray(decoder_w, f32)),
        ('bdec', jnp.asarray(decoder_b, f32).T),
        ('betaT', beta.T),
    ]
    slab, off = _pack_slab(entries)
    w_rows = slab.shape[0]

    def padg(a, fill=0):
        if Gp == G:
            return a
        pad_shape = (Gp - G,) + a.shape[1:]
        return jnp.concatenate([a, jnp.full(pad_shape, fill, a.dtype)], axis=0)

    def rows_n(a):
        return a.reshape(steps, 1, LANES)

    def rows_e(a):
        a = a.reshape(Gp // GPS, LE)
        a = jnp.concatenate(
            [a, jnp.zeros((Gp // GPS, 128 - LE), a.dtype)], axis=1)
        return a.reshape(steps, 1, LANES)

    src = jnp.asarray(edge_index[:, 0, :], i32)
    idx_x_i = jnp.asarray(idx_x, i32)
    idxx3 = rows_n(padg(idx_x_i))
    idxs3 = rows_e(padg(jnp.take_along_axis(idx_x_i, src, axis=1)))
    idxw3 = rows_n(padg(jnp.asarray(idx_w, f32)))
    xb3 = rows_n(padg(jnp.asarray(x_batch, i32)))
    dst3 = rows_e(padg(jnp.asarray(edge_index[:, 1, :], i32)))
    ew3 = rows_e(padg(jnp.asarray(edge_w, f32)))
    docT = padg(jnp.asarray(doc_input, f32)).reshape(Gp * B, V).T

    kern = functools.partial(_fused_kernel, off)
    thetaT, scal = pl.pallas_call(
        kern,
        out_shape=[jax.ShapeDtypeStruct((K, Gp * B), f32),
                   jax.ShapeDtypeStruct((4, Gp), f32)],
        grid_spec=pltpu.PrefetchScalarGridSpec(
            num_scalar_prefetch=0,
            grid=(steps,),
            in_specs=[
                pl.BlockSpec((w_rows, 128), lambda g: (0, 0)),
                pl.BlockSpec((1, 1, LANES), lambda g: (g, 0, 0)),
                pl.BlockSpec((1, 1, LANES), lambda g: (g, 0, 0)),
                pl.BlockSpec((1, 1, LANES), lambda g: (g, 0, 0)),
                pl.BlockSpec((1, 1, LANES), lambda g: (g, 0, 0)),
                pl.BlockSpec((1, 1, LANES), lambda g: (g, 0, 0)),
                pl.BlockSpec((1, 1, LANES), lambda g: (g, 0, 0)),
                pl.BlockSpec((V, D_STEP), lambda g: (0, g)),
            ],
            out_specs=[
                pl.BlockSpec((K, D_STEP), lambda g: (0, g)),
                pl.BlockSpec((4, GB), lambda g: (0, g)),
            ]),
        compiler_params=pltpu.CompilerParams(
            dimension_semantics=("parallel",),
            vmem_limit_bytes=50 * 1024 * 1024),
    )(slab, idxx3, idxs3, idxw3, xb3, dst3, ew3, docT)

    theta = thetaT.T.reshape(Gp, B, K)[:G]
    loss = scal[0, :G]
    outputs = {'loss': loss,
               'recon_word': scal[1, :G],
               'KL1': scal[2, :G],
               'KL2': scal[3, :G],
               'recon_structure': jnp.zeros_like(loss)}
    if not batched:
        outputs = jax.tree_util.tree_map(lambda a: a[0], outputs)
        theta, loss = theta[0], loss[0]
    return outputs, theta, loss


# speedup vs baseline: 1.4634x; 1.0381x over previous
import functools

import jax
import jax.numpy as jnp
from jax.experimental import pallas as pl
from jax.experimental.pallas import tpu as pltpu

B = 2
V = 32
NI = 16
NW = 16
H = 32
K = 8
N_NODES = 16
E = 12
BN_EPS = 1e-5
BN_SCALE = (1.0 + BN_EPS) ** -0.5

GPS = 8
SG_PER_STEP = 256
GB = GPS * SG_PER_STEP
D_STEP = GB * B
LN = GPS * N_NODES
LE = GPS * E
DSG = GPS * B
LANES = SG_PER_STEP * 128


def _pad8(n):
    return ((n + 7) // 8) * 8


def _pack_slab(entries):
    off = {}
    row = 0
    pieces = []
    for name, a in entries:
        h, w = a.shape
        hp = _pad8(h)
        off[name] = (row, h, w)
        row += hp
        pieces.append(jnp.pad(a.astype(jnp.float32),
                              ((0, hp - h), (0, 128 - w))))
    return jnp.concatenate(pieces, axis=0), off


def _fused_kernel(off, slab_ref, idxx_ref, idxs_ref, idxw_ref, xb_ref,
                  dst_ref, ew_ref, docT_ref, theta_ref, scal_ref):
    f32 = jnp.float32
    bf16 = jnp.bfloat16
    i32 = jnp.int32
    iota = jax.lax.broadcasted_iota
    nt = (((1,), (1,)), ((), ()))

    def W(name):
        r, h, w = off[name]
        return slab_ref[r:r + h, 0:w]

    def bdot(a, b):
        return jnp.dot(a, b, preferred_element_type=f32)

    idxx = idxx_ref[0]
    oh_x = (iota(i32, (V, LANES), 0) == idxx).astype(f32)
    xT = jnp.dot(W('wvT'), oh_x, preferred_element_type=f32)

    idxs = idxs_ref[0]
    oh_s = (iota(i32, (V, LANES), 0) == idxs).astype(f32)
    x_srcT = jnp.dot(W('wvT'), oh_s, preferred_element_type=f32)
    x_srcT = x_srcT * ew_ref[0]

    eoff = iota(i32, (1, 128), 1) // E * N_NODES
    agg_parts = []
    for sg in range(SG_PER_STEP):
        lo, hi = sg * 128, (sg + 1) * 128
        gdst = dst_ref[0][:, lo:hi] + eoff
        mdstT = (iota(i32, (LN, 128), 0) == gdst).astype(f32)
        agg_parts.append(
            jax.lax.dot_general(x_srcT[:, lo:hi], mdstT, nt,
                                preferred_element_type=f32))
    aggT = jnp.concatenate(agg_parts, axis=1)
    aggT = aggT + idxw_ref[0] * xT

    ax = jnp.concatenate([aggT, xT], axis=0)
    gnnT = bdot(W('Wrr'), ax) + W('b_gnn')
    enc1T = jnp.tanh(gnnT * BN_SCALE)
    ex = jnp.concatenate([enc1T, xT], axis=0)
    pre = bdot(W('W12'), ex) + W('b12')
    gT = jax.nn.sigmoid(pre[0:H]) * jnp.tanh(pre[H:2 * H])

    doff = iota(i32, (1, 128), 1) // N_NODES * B
    enc2_parts = []
    for sg in range(SG_PER_STEP):
        lo, hi = sg * 128, (sg + 1) * 128
        gdoc = xb_ref[0][:, lo:hi] + doff
        mselT = (iota(i32, (DSG, 128), 0) == gdoc).astype(f32)
        enc2_parts.append(
            jax.lax.dot_general(gT[:, lo:hi], mselT, nt,
                                preferred_element_type=f32))
    enc2T = jnp.concatenate(enc2_parts, axis=1)

    gm = bdot(W('Wg'), enc2T) + W('bg')
    post_mean = gm[0:K] * BN_SCALE
    post_logvar = gm[K:2 * K]

    docT = docT_ref[...]
    h1 = jnp.tanh(bdot(W('We1'), docT) + W('be1'))
    h2 = jnp.tanh(bdot(W('We2'), h1) + W('be2'))
    pp = bdot(W('Wp'), h2) + W('bp')
    prior_mean = pp[0:K]
    prior_logvar = pp[K:2 * K]

    td = bdot(W('Wdec'), prior_mean) + W('bdec')
    e = jnp.exp(td - jnp.max(td, axis=0, keepdims=True))
    theta = e / jnp.sum(e, axis=0, keepdims=True)
    recon = jnp.dot(W('betaT'), theta, preferred_element_type=f32)
    nl = -jnp.sum(docT * jnp.log(recon + 1e-10), axis=0, keepdims=True)

    post_var = jnp.exp(post_logvar)
    prior_var = jnp.exp(prior_logvar)
    kl1 = 0.5 * jnp.sum(
        prior_logvar - post_logvar
        + (post_var + (post_mean - prior_mean) ** 2) / prior_var - 1.0,
        axis=0, keepdims=True)
    kl2 = -0.5 * jnp.sum(1.0 - post_mean ** 2 + post_logvar - post_var,
                         axis=0, keepdims=True)

    quad = jnp.concatenate([nl + kl1 + kl2, nl, kl1, kl2], axis=0)
    pool = (iota(i32, (D_STEP, GB), 0) // B
            == iota(i32, (D_STEP, GB), 1)).astype(f32)
    scal_ref[...] = jnp.dot(quad, pool,
                            preferred_element_type=f32) * (1.0 / B)
    theta_ref[...] = theta.T


def kernel(word_vec, gnn_wrel, gnn_brel, gnn_wroot, enc2_fc1_w, enc2_fc1_b,
           enc2_fc2_w, enc2_fc2_b, ge_mean_w, ge_mean_b, ge_logvar_w,
           ge_logvar_b, enc1_fc_w, enc1_fc_b, enc2_fc_w, enc2_fc_b,
           mean_fc_w, mean_fc_b, logvar_fc_w, logvar_fc_b, decoder_w,
           decoder_b, topic_vec, idx_x, idx_w, x_batch, edge_index,
           edge_w, doc_input):
    f32 = jnp.float32
    i32 = jnp.int32

    batched = doc_input.ndim == 3
    if not batched:
        idx_x, idx_w, x_batch = idx_x[None], idx_w[None], x_batch[None]
        edge_index, edge_w, doc_input = (edge_index[None], edge_w[None],
                                         doc_input[None])
    G = doc_input.shape[0]
    Gp = ((G + GB - 1) // GB) * GB
    steps = Gp // GB

    beta = jax.nn.softmax(
        jnp.asarray(topic_vec, f32) @ jnp.asarray(word_vec, f32).T, axis=-1)
    w1 = jnp.asarray(enc2_fc1_w, f32)
    w2 = jnp.asarray(enc2_fc2_w, f32)
    entries = [
        ('wvT', jnp.asarray(word_vec, f32).T),
        ('Wrr', jnp.concatenate([jnp.asarray(gnn_wrel, f32),
                                 jnp.asarray(gnn_wroot, f32)], axis=1)),
        ('b_gnn', jnp.asarray(gnn_brel, f32).T),
        ('W12', jnp.concatenate(
            [jnp.concatenate([w1[:, :NW], w2[:, :NW]], axis=0),
             jnp.concatenate([w1[:, NW:], w2[:, NW:]], axis=0)], axis=1)),
        ('b12', jnp.concatenate([jnp.asarray(enc2_fc1_b, f32),
                                 jnp.asarray(enc2_fc2_b, f32)], axis=1).T),
        ('Wg', jnp.concatenate([jnp.asarray(ge_mean_w, f32),
                                jnp.asarray(ge_logvar_w, f32)], axis=0)),
        ('bg', jnp.concatenate([jnp.asarray(ge_mean_b, f32),
                                jnp.asarray(ge_logvar_b, f32)], axis=1).T),
        ('We1', jnp.asarray(enc1_fc_w, f32)),
        ('be1', jnp.asarray(enc1_fc_b, f32).T),
        ('We2', jnp.asarray(enc2_fc_w, f32)),
        ('be2', jnp.asarray(enc2_fc_b, f32).T),
        ('Wp', jnp.concatenate([jnp.asarray(mean_fc_w, f32),
                                jnp.asarray(logvar_fc_w, f32)], axis=0)),
        ('bp', jnp.concatenate([jnp.asarray(mean_fc_b, f32),
                                jnp.asarray(logvar_fc_b, f32)], axis=1).T),
        ('Wdec', jnp.asarray(decoder_w, f32)),
        ('bdec', jnp.asarray(decoder_b, f32).T),
        ('betaT', beta.T),
    ]
    slab, off = _pack_slab(entries)
    w_rows = slab.shape[0]

    def padg(a, fill=0):
        if Gp == G:
            return a
        pad_shape = (Gp - G,) + a.shape[1:]
        return jnp.concatenate([a, jnp.full(pad_shape, fill, a.dtype)], axis=0)

    def rows_n(a):
        return a.reshape(steps, 1, LANES)

    def rows_e(a):
        a = a.reshape(Gp // GPS, LE)
        a = jnp.concatenate(
            [a, jnp.zeros((Gp // GPS, 128 - LE), a.dtype)], axis=1)
        return a.reshape(steps, 1, LANES)

    src = jnp.asarray(edge_index[:, 0, :], i32)
    idx_x_i = jnp.asarray(idx_x, i32)
    idxx3 = rows_n(padg(idx_x_i))
    idxs3 = rows_e(padg(jnp.take_along_axis(idx_x_i, src, axis=1)))
    idxw3 = rows_n(padg(jnp.asarray(idx_w, f32)))
    xb3 = rows_n(padg(jnp.asarray(x_batch, i32)))
    dst3 = rows_e(padg(jnp.asarray(edge_index[:, 1, :], i32)))
    ew3 = rows_e(padg(jnp.asarray(edge_w, f32)))
    docT = padg(jnp.asarray(doc_input, f32)).reshape(Gp * B, V).T

    kern = functools.partial(_fused_kernel, off)
    thetaT, scal = pl.pallas_call(
        kern,
        out_shape=[jax.ShapeDtypeStruct((Gp * B, K), f32),
                   jax.ShapeDtypeStruct((4, Gp), f32)],
        grid_spec=pltpu.PrefetchScalarGridSpec(
            num_scalar_prefetch=0,
            grid=(steps,),
            in_specs=[
                pl.BlockSpec((w_rows, 128), lambda g: (0, 0)),
                pl.BlockSpec((1, 1, LANES), lambda g: (g, 0, 0)),
                pl.BlockSpec((1, 1, LANES), lambda g: (g, 0, 0)),
                pl.BlockSpec((1, 1, LANES), lambda g: (g, 0, 0)),
                pl.BlockSpec((1, 1, LANES), lambda g: (g, 0, 0)),
                pl.BlockSpec((1, 1, LANES), lambda g: (g, 0, 0)),
                pl.BlockSpec((1, 1, LANES), lambda g: (g, 0, 0)),
                pl.BlockSpec((V, D_STEP), lambda g: (0, g)),
            ],
            out_specs=[
                pl.BlockSpec((D_STEP, K), lambda g: (g, 0)),
                pl.BlockSpec((4, GB), lambda g: (0, g)),
            ]),
        compiler_params=pltpu.CompilerParams(
            dimension_semantics=("parallel",),
            vmem_limit_bytes=50 * 1024 * 1024),
    )(slab, idxx3, idxs3, idxw3, xb3, dst3, ew3, docT)

    theta = thetaT.reshape(Gp, B, K)[:G]
    loss = scal[0, :G]
    outputs = {'loss': loss,
               'recon_word': scal[1, :G],
               'KL1': scal[2, :G],
               'KL2': scal[3, :G],
               'recon_structure': jnp.zeros_like(loss)}
    if not batched:
        outputs = jax.tree_util.tree_map(lambda a: a[0], outputs)
        theta, loss = theta[0], loss[0]
    return outputs, theta, loss


# speedup vs baseline: 1.4890x; 1.0175x over previous
import functools

import jax
import jax.numpy as jnp
from jax.experimental import pallas as pl
from jax.experimental.pallas import tpu as pltpu

B = 2
V = 32
NI = 16
NW = 16
H = 32
K = 8
N_NODES = 16
E = 12
BN_EPS = 1e-5
BN_SCALE = (1.0 + BN_EPS) ** -0.5

GPS = 8
SG_PER_STEP = 256
GB = GPS * SG_PER_STEP
D_STEP = GB * B
LN = GPS * N_NODES
LE = GPS * E
DSG = GPS * B
LANES = SG_PER_STEP * 128


def _pad8(n):
    return ((n + 7) // 8) * 8


def _pack_slab(entries):
    off = {}
    row = 0
    pieces = []
    for name, a in entries:
        h, w = a.shape
        hp = _pad8(h)
        off[name] = (row, h, w)
        row += hp
        pieces.append(jnp.pad(a.astype(jnp.float32),
                              ((0, hp - h), (0, 128 - w))))
    return jnp.concatenate(pieces, axis=0), off


def _fused_kernel(off, slab_ref, idxx_ref, idxs_ref, idxw_ref, xb_ref,
                  dst_ref, ew_ref, doc_ref, theta_ref, scal_ref):
    f32 = jnp.float32
    bf16 = jnp.bfloat16
    i32 = jnp.int32
    iota = jax.lax.broadcasted_iota
    nt = (((1,), (1,)), ((), ()))

    def W(name):
        r, h, w = off[name]
        return slab_ref[r:r + h, 0:w]

    def bdot(a, b):
        return jnp.dot(a, b, preferred_element_type=f32)

    idxx = idxx_ref[0]
    oh_x = (iota(i32, (V, LANES), 0) == idxx).astype(f32)
    xT = jnp.dot(W('wvT'), oh_x, preferred_element_type=f32)

    idxs = idxs_ref[0]
    oh_s = (iota(i32, (V, LANES), 0) == idxs).astype(f32)
    x_srcT = jnp.dot(W('wvT'), oh_s, preferred_element_type=f32)
    x_srcT = x_srcT * ew_ref[0]

    eoff = iota(i32, (1, 128), 1) // E * N_NODES
    agg_parts = []
    for sg in range(SG_PER_STEP):
        lo, hi = sg * 128, (sg + 1) * 128
        gdst = dst_ref[0][:, lo:hi] + eoff
        mdstT = (iota(i32, (LN, 128), 0) == gdst).astype(f32)
        agg_parts.append(
            jax.lax.dot_general(x_srcT[:, lo:hi], mdstT, nt,
                                preferred_element_type=f32))
    aggT = jnp.concatenate(agg_parts, axis=1)
    aggT = aggT + idxw_ref[0] * xT

    ax = jnp.concatenate([aggT, xT], axis=0)
    gnnT = bdot(W('Wrr'), ax) + W('b_gnn')
    enc1T = jnp.tanh(gnnT * BN_SCALE)
    ex = jnp.concatenate([enc1T, xT], axis=0)
    pre = bdot(W('W12'), ex) + W('b12')
    gT = jax.nn.sigmoid(pre[0:H]) * jnp.tanh(pre[H:2 * H])

    doff = iota(i32, (1, 128), 1) // N_NODES * B
    enc2_parts = []
    for sg in range(SG_PER_STEP):
        lo, hi = sg * 128, (sg + 1) * 128
        gdoc = xb_ref[0][:, lo:hi] + doff
        mselT = (iota(i32, (DSG, 128), 0) == gdoc).astype(f32)
        enc2_parts.append(
            jax.lax.dot_general(gT[:, lo:hi], mselT, nt,
                                preferred_element_type=f32))
    enc2T = jnp.concatenate(enc2_parts, axis=1)

    gm = bdot(W('Wg'), enc2T) + W('bg')
    post_mean = gm[0:K] * BN_SCALE
    post_logvar = gm[K:2 * K]

    docT = doc_ref[...].T
    h1 = jnp.tanh(bdot(W('We1'), docT) + W('be1'))
    h2 = jnp.tanh(bdot(W('We2'), h1) + W('be2'))
    pp = bdot(W('Wp'), h2) + W('bp')
    prior_mean = pp[0:K]
    prior_logvar = pp[K:2 * K]

    td = bdot(W('Wdec'), prior_mean) + W('bdec')
    e = jnp.exp(td - jnp.max(td, axis=0, keepdims=True))
    theta = e / jnp.sum(e, axis=0, keepdims=True)
    recon = jnp.dot(W('betaT'), theta, preferred_element_type=f32)
    nl = -jnp.sum(docT * jnp.log(recon + 1e-10), axis=0, keepdims=True)

    post_var = jnp.exp(post_logvar)
    prior_var = jnp.exp(prior_logvar)
    kl1 = 0.5 * jnp.sum(
        prior_logvar - post_logvar
        + (post_var + (post_mean - prior_mean) ** 2) / prior_var - 1.0,
        axis=0, keepdims=True)
    kl2 = -0.5 * jnp.sum(1.0 - post_mean ** 2 + post_logvar - post_var,
                         axis=0, keepdims=True)

    quad = jnp.concatenate([nl + kl1 + kl2, nl, kl1, kl2], axis=0)
    pool = (iota(i32, (D_STEP, GB), 0) // B
            == iota(i32, (D_STEP, GB), 1)).astype(f32)
    scal_ref[...] = jnp.dot(quad, pool,
                            preferred_element_type=f32) * (1.0 / B)
    theta_ref[...] = theta.T


def kernel(word_vec, gnn_wrel, gnn_brel, gnn_wroot, enc2_fc1_w, enc2_fc1_b,
           enc2_fc2_w, enc2_fc2_b, ge_mean_w, ge_mean_b, ge_logvar_w,
           ge_logvar_b, enc1_fc_w, enc1_fc_b, enc2_fc_w, enc2_fc_b,
           mean_fc_w, mean_fc_b, logvar_fc_w, logvar_fc_b, decoder_w,
           decoder_b, topic_vec, idx_x, idx_w, x_batch, edge_index,
           edge_w, doc_input):
    f32 = jnp.float32
    i32 = jnp.int32

    batched = doc_input.ndim == 3
    if not batched:
        idx_x, idx_w, x_batch = idx_x[None], idx_w[None], x_batch[None]
        edge_index, edge_w, doc_input = (edge_index[None], edge_w[None],
                                         doc_input[None])
    G = doc_input.shape[0]
    Gp = ((G + GB - 1) // GB) * GB
    steps = Gp // GB

    beta = jax.nn.softmax(
        jnp.asarray(topic_vec, f32) @ jnp.asarray(word_vec, f32).T, axis=-1)
    w1 = jnp.asarray(enc2_fc1_w, f32)
    w2 = jnp.asarray(enc2_fc2_w, f32)
    entries = [
        ('wvT', jnp.asarray(word_vec, f32).T),
        ('Wrr', jnp.concatenate([jnp.asarray(gnn_wrel, f32),
                                 jnp.asarray(gnn_wroot, f32)], axis=1)),
        ('b_gnn', jnp.asarray(gnn_brel, f32).T),
        ('W12', jnp.concatenate(
            [jnp.concatenate([w1[:, :NW], w2[:, :NW]], axis=0),
             jnp.concatenate([w1[:, NW:], w2[:, NW:]], axis=0)], axis=1)),
        ('b12', jnp.concatenate([jnp.asarray(enc2_fc1_b, f32),
                                 jnp.asarray(enc2_fc2_b, f32)], axis=1).T),
        ('Wg', jnp.concatenate([jnp.asarray(ge_mean_w, f32),
                                jnp.asarray(ge_logvar_w, f32)], axis=0)),
        ('bg', jnp.concatenate([jnp.asarray(ge_mean_b, f32),
                                jnp.asarray(ge_logvar_b, f32)], axis=1).T),
        ('We1', jnp.asarray(enc1_fc_w, f32)),
        ('be1', jnp.asarray(enc1_fc_b, f32).T),
        ('We2', jnp.asarray(enc2_fc_w, f32)),
        ('be2', jnp.asarray(enc2_fc_b, f32).T),
        ('Wp', jnp.concatenate([jnp.asarray(mean_fc_w, f32),
                                jnp.asarray(logvar_fc_w, f32)], axis=0)),
        ('bp', jnp.concatenate([jnp.asarray(mean_fc_b, f32),
                                jnp.asarray(logvar_fc_b, f32)], axis=1).T),
        ('Wdec', jnp.asarray(decoder_w, f32)),
        ('bdec', jnp.asarray(decoder_b, f32).T),
        ('betaT', beta.T),
    ]
    slab, off = _pack_slab(entries)
    w_rows = slab.shape[0]

    def padg(a, fill=0):
        if Gp == G:
            return a
        pad_shape = (Gp - G,) + a.shape[1:]
        return jnp.concatenate([a, jnp.full(pad_shape, fill, a.dtype)], axis=0)

    def rows_n(a):
        return a.reshape(steps, 1, LANES)

    def rows_e(a):
        a = a.reshape(Gp // GPS, LE)
        a = jnp.concatenate(
            [a, jnp.zeros((Gp // GPS, 128 - LE), a.dtype)], axis=1)
        return a.reshape(steps, 1, LANES)

    src = jnp.asarray(edge_index[:, 0, :], i32)
    idx_x_i = jnp.asarray(idx_x, i32)
    idxx3 = rows_n(padg(idx_x_i))
    idxs3 = rows_e(padg(jnp.take_along_axis(idx_x_i, src, axis=1)))
    idxw3 = rows_n(padg(jnp.asarray(idx_w, f32)))
    xb3 = rows_n(padg(jnp.asarray(x_batch, i32)))
    dst3 = rows_e(padg(jnp.asarray(edge_index[:, 1, :], i32)))
    ew3 = rows_e(padg(jnp.asarray(edge_w, f32)))
    doc2 = padg(jnp.asarray(doc_input, f32)).reshape(Gp * B, V)

    kern = functools.partial(_fused_kernel, off)
    thetaT, scal = pl.pallas_call(
        kern,
        out_shape=[jax.ShapeDtypeStruct((Gp * B, K), f32),
                   jax.ShapeDtypeStruct((4, Gp), f32)],
        grid_spec=pltpu.PrefetchScalarGridSpec(
            num_scalar_prefetch=0,
            grid=(steps,),
            in_specs=[
                pl.BlockSpec((w_rows, 128), lambda g: (0, 0)),
                pl.BlockSpec((1, 1, LANES), lambda g: (g, 0, 0)),
                pl.BlockSpec((1, 1, LANES), lambda g: (g, 0, 0)),
                pl.BlockSpec((1, 1, LANES), lambda g: (g, 0, 0)),
                pl.BlockSpec((1, 1, LANES), lambda g: (g, 0, 0)),
                pl.BlockSpec((1, 1, LANES), lambda g: (g, 0, 0)),
                pl.BlockSpec((1, 1, LANES), lambda g: (g, 0, 0)),
                pl.BlockSpec((D_STEP, V), lambda g: (g, 0)),
            ],
            out_specs=[
                pl.BlockSpec((D_STEP, K), lambda g: (g, 0)),
                pl.BlockSpec((4, GB), lambda g: (0, g)),
            ]),
        compiler_params=pltpu.CompilerParams(
            dimension_semantics=("parallel",),
            vmem_limit_bytes=50 * 1024 * 1024),
    )(slab, idxx3, idxs3, idxw3, xb3, dst3, ew3, doc2)

    theta = thetaT.reshape(Gp, B, K)[:G]
    loss = scal[0, :G]
    outputs = {'loss': loss,
               'recon_word': scal[1, :G],
               'KL1': scal[2, :G],
               'KL2': scal[3, :G],
               'recon_structure': jnp.zeros_like(loss)}
    if not batched:
        outputs = jax.tree_util.tree_map(lambda a: a[0], outputs)
        theta, loss = theta[0], loss[0]
    return outputs, theta, loss
